# baseline (speedup 1.0000x reference)
import functools

import jax
import jax.numpy as jnp
from jax import lax
from jax.experimental import pallas as pl
from jax.experimental.pallas import tpu as pltpu
from jax.experimental.pallas import tpu_sc as plsc

B, L = 2, 2048
DM, DC, DK = 1024, 768, 64
D = DM + DC + DK
E, H, K = 8, 512, 2
HG = max(D // 2, 128)
T = B * L
N = T * K
TB = 512
BLK = 1024
NP = N + E * BLK
NBLK = NP // BLK
NW = 32
CH = N // NW // 64
DH = 1024
D1 = 1792


def _ln(x, g, b):
    mu = x.mean(-1, keepdims=True)
    v = ((x - mu) ** 2).mean(-1, keepdims=True)
    return (x - mu) * lax.rsqrt(v + 1e-5) * g + b


def _pack_pair(hi_f, lo_f):
    uh = lax.bitcast_convert_type(hi_f, jnp.uint32)
    ul = lax.bitcast_convert_type(lo_f, jnp.uint32)
    return lax.bitcast_convert_type(
        (uh & jnp.uint32(0xFFFF0000)) | (ul >> 16), jnp.int32)


def _gate_kernel(id_ref, ct_ref, cb_ref,
                 gW1a_ref, gW1b_ref, gb1_ref, glng_ref, glnb_ref,
                 gW2_ref, gb2_ref, gW3_ref, gb3_ref,
                 xp_ref, pos_ref, be_ref, w_ref, ti_acc):
    idv = id_ref[...]
    ctv = ct_ref[...]
    cbv = cb_ref[...]
    hi_f = jnp.concatenate(
        [ctv, cbv, jnp.zeros((TB, DH - DC - DK), jnp.float32)], axis=-1)
    xp_ref[...] = _pack_pair(hi_f, idv)
    xa = jnp.concatenate([idv, ctv], axis=-1).astype(jnp.bfloat16)
    h = jnp.dot(xa, gW1a_ref[...], preferred_element_type=jnp.float32)
    h += jnp.dot(cbv.astype(jnp.bfloat16), gW1b_ref[...],
                 preferred_element_type=jnp.float32)
    h = jax.nn.relu(_ln(h + gb1_ref[...], glng_ref[...], glnb_ref[...]))
    h2 = jnp.dot(h.astype(jnp.bfloat16), gW2_ref[...],
                 preferred_element_type=jnp.float32) + gb2_ref[...]
    h2 = jax.nn.relu(h2)
    logits = jnp.dot(h2.astype(jnp.bfloat16), gW3_ref[...],
                     preferred_element_type=jnp.float32) + gb3_ref[...]
    probs = jax.nn.softmax(logits, axis=-1)
    eidx = lax.broadcasted_iota(jnp.int32, (TB, E), 1)
    i1 = jnp.argmax(probs, axis=-1).astype(jnp.int32)
    p1 = jnp.max(probs, axis=-1)
    masked = jnp.where(eidx == i1[:, None], -jnp.inf, probs)
    i2 = jnp.argmax(masked, axis=-1).astype(jnp.int32)
    p2 = jnp.max(masked, axis=-1)
    denom = p1 + p2 + 1e-8
    w_ref[...] = jnp.stack([p1 / denom, p2 / denom], axis=-1)
    i = pl.program_id(0)
    ti_acc[pl.ds(i * (TB // 128), TB // 128)] = i1.reshape(TB // 128, 128)
    ti_acc[pl.ds(T // 128 + i * (TB // 128), TB // 128)] = (
        i2.reshape(TB // 128, 128))

    @pl.when(i == T // TB - 1)
    def _route():
        _route_body(ti_acc[...], pos_ref, be_ref)


def _route_body(ei, pos_ref, be_ref):
    cU = (lax.broadcasted_iota(jnp.int32, (128, 128), 0) <
          lax.broadcasted_iota(jnp.int32, (128, 128), 1)).astype(jnp.bfloat16)
    L64 = (lax.broadcasted_iota(jnp.int32, (64, 64), 1) <
           lax.broadcasted_iota(jnp.int32, (64, 64), 0)).astype(jnp.bfloat16)
    pos_f = jnp.zeros((64, 128), jnp.float32)
    seg = jnp.int32(0)
    seg_ends = []
    for e in range(E):
        m = ei == e
        mb = m.astype(jnp.bfloat16)
        prefix = jnp.dot(mb, cU, preferred_element_type=jnp.float32)
        rowsum = prefix[:, 127:128] + m.astype(jnp.float32)[:, 127:128]
        carry = jnp.dot(L64, rowsum.astype(jnp.bfloat16),
                        preferred_element_type=jnp.float32)
        rank = prefix + carry
        pos_f = pos_f + jnp.where(m, seg.astype(jnp.float32) + rank, 0.0)
        cnt = jnp.sum(m.astype(jnp.float32)).astype(jnp.int32)
        pe = ((cnt + BLK - 1) // BLK) * BLK
        seg = seg + pe
        seg_ends.append(seg)
    lane = lax.broadcasted_iota(jnp.int32, (1, 128), 1)
    bi = lane * BLK
    be = jnp.zeros((1, 128), jnp.int32)
    for e in range(E):
        be = be + (bi >= seg_ends[e]).astype(jnp.int32)
    be_ref[...] = jnp.where(lane == NBLK, seg_ends[-1],
                            jnp.minimum(be, E - 1))
    pos_ref[...] = pos_f.astype(jnp.int32)


_vmesh = plsc.VectorSubcoreMesh(core_axis_name="c", subcore_axis_name="s")


SCH = 8


@functools.partial(
    pl.kernel, mesh=_vmesh,
    out_type=jax.ShapeDtypeStruct((NP, DH), jnp.int32),
    scratch_types=[pltpu.VMEM((SCH, 32), jnp.int32),
                   pltpu.VMEM((32, DH), jnp.int32),
                   pltpu.VMEM((32, DH), jnp.int32),
                   pltpu.SemaphoreType.DMA,
                   pltpu.SemaphoreType.DMA])
def _scatter_x(x_hbm, pos_hbm, xs_hbm, pos_v, xbuf0, xbuf1, sem0, sem1):
    wid = lax.axis_index("s") * 2 + lax.axis_index("c")
    t0 = (wid * 256) % T
    pltpu.sync_copy(pos_hbm.at[wid], pos_v)
    bufs = (xbuf0, xbuf1)
    sems = (sem0, sem1)
    pending = [None, None]
    for q in range(SCH):
        b = bufs[q % 2]
        if pending[q % 2] is not None:
            pending[q % 2].wait()
        pltpu.sync_copy(x_hbm.at[pl.ds(t0 + q * 32, 32)], b)
        pending[q % 2] = pltpu.async_copy(b, xs_hbm.at[pos_v.at[q]],
                                          sems[q % 2])
    pending[0].wait()
    pending[1].wait()


@functools.partial(
    pl.kernel, mesh=_vmesh,
    out_type=jax.ShapeDtypeStruct((N, H // 4), jnp.float32),
    scratch_types=[pltpu.VMEM((CH, 64), jnp.int32),
                   pltpu.VMEM((64, H // 4), jnp.float32),
                   pltpu.SemaphoreType.DMA])
def _gather_o(os_hbm, pos_hbm, sel_hbm, pos_v, obuf, sem):
    wid = lax.axis_index("s") * 2 + lax.axis_index("c")
    base = wid * 256
    pltpu.sync_copy(pos_hbm.at[wid], pos_v)
    for c in range(CH):
        pltpu.async_copy(os_hbm.at[pos_v.at[c]], obuf, sem).wait()
        pltpu.sync_copy(obuf, sel_hbm.at[pl.ds(base + c * 64, 64)])


def _expert_kernel(be_ref, xs_ref, eW1a_ref, eW1b_ref, eb1_ref,
                   g1_ref, b1_ref, eW2_ref, eb2_ref, eW3_ref, eb3_ref,
                   g2_ref, b2_ref, os_ref):
    @pl.when(pl.program_id(0) * BLK < be_ref[NBLK])
    def _body():
        xi = lax.bitcast_convert_type(xs_ref[...], jnp.uint32)
        lo = lax.bitcast_convert_type(xi << 16, jnp.float32)
        hi = lax.bitcast_convert_type(xi & jnp.uint32(0xFFFF0000), jnp.float32)
        xa = lo.astype(jnp.bfloat16)
        xb = hi.astype(jnp.bfloat16)
        a = jnp.dot(xa, eW1a_ref[0], preferred_element_type=jnp.float32)
        a += jnp.dot(xb, eW1b_ref[0], preferred_element_type=jnp.float32)
        a = jax.nn.relu(_ln(a + eb1_ref[0], g1_ref[0], b1_ref[0]))
        b2v = jnp.dot(a.astype(jnp.bfloat16), eW2_ref[0],
                      preferred_element_type=jnp.float32) + eb2_ref[0]
        b2v = jax.nn.relu(b2v)
        o = jnp.dot(b2v.astype(jnp.bfloat16), eW3_ref[0],
                    preferred_element_type=jnp.float32) + eb3_ref[0]
        os_ref[...] = _ln(o, g2_ref[0], b2_ref[0])


def _out_kernel(sel0_ref, sel1_ref, w_ref, id_ref, Wo_ref, bo_ref,
                alpha_ref, out_ref):
    w = w_ref[...]
    fused = sel0_ref[...] * w[:, 0:1] + sel1_ref[...] * w[:, 1:2]
    proj = jnp.dot(fused.astype(jnp.bfloat16), Wo_ref[...],
                   preferred_element_type=jnp.float32) + bo_ref[...]
    out_ref[...] = id_ref[...] + alpha_ref[0, 0] * proj


def kernel(id_emb, content_emb, collab_emb, params):
    p = params
    bf = jnp.bfloat16
    row = lambda a: a.reshape(1, -1)
    id_flat = id_emb.reshape(T, DM)
    ct_flat = content_emb.reshape(T, DC)
    cb_flat = collab_emb.reshape(T, DK)

    def const_spec(shape):
        return pl.BlockSpec(shape, lambda i: (0,) * len(shape))

    gW1a = p['gW1'][:D1].astype(bf)
    gW1b = p['gW1'][D1:].astype(bf)
    gate_ops = [id_flat, ct_flat, cb_flat, gW1a, gW1b, row(p['gb1']),
                row(p['gln_g']), row(p['gln_b']), p['gW2'].astype(bf),
                row(p['gb2']), p['gW3'].astype(bf), row(p['gb3'])]
    xp, pos64, be128, w = pl.pallas_call(
        _gate_kernel,
        grid=(T // TB,),
        in_specs=[pl.BlockSpec((TB, DM), lambda i: (i, 0)),
                  pl.BlockSpec((TB, DC), lambda i: (i, 0)),
                  pl.BlockSpec((TB, DK), lambda i: (i, 0))] +
                 [const_spec(op.shape) for op in gate_ops[3:]],
        out_specs=[pl.BlockSpec((TB, DH), lambda i: (i, 0)),
                   const_spec((64, 128)),
                   const_spec((1, 128)),
                   pl.BlockSpec((TB, K), lambda i: (i, 0))],
        out_shape=[jax.ShapeDtypeStruct((T, DH), jnp.int32),
                   jax.ShapeDtypeStruct((64, 128), jnp.int32),
                   jax.ShapeDtypeStruct((1, 128), jnp.int32),
                   jax.ShapeDtypeStruct((T, K), jnp.float32)],
        scratch_shapes=[pltpu.VMEM((64, 128), jnp.int32)],
    )(*gate_ops)
    pos3 = pos64.reshape(NW, CH, 64)
    be = be128[0, :NBLK + 1]

    xs = _scatter_x(xp, pos64.reshape(NW, SCH, 32))

    e3 = lambda a: a.reshape(E, 1, -1)
    eW1a = p['eW1'][:, :DM].astype(bf)
    eW1b = jnp.concatenate(
        [p['eW1'][:, DM:].astype(bf), jnp.zeros((E, DH - DC - DK, H), bf)],
        axis=1)
    ew_ops = [xs, eW1a, eW1b, e3(p['eb1']), e3(p['eln1_g']), e3(p['eln1_b']),
              p['eW2'].astype(bf), e3(p['eb2']), p['eW3'].astype(bf),
              e3(p['eb3']), e3(p['eln2_g']), e3(p['eln2_b'])]
    grid_spec = pltpu.PrefetchScalarGridSpec(
        num_scalar_prefetch=1,
        grid=(NBLK,),
        in_specs=[
            pl.BlockSpec((BLK, DH), lambda i, be_r: (i, 0)),
            pl.BlockSpec((1, DH, H), lambda i, be_r: (be_r[i], 0, 0)),
            pl.BlockSpec((1, DH, H), lambda i, be_r: (be_r[i], 0, 0)),
            pl.BlockSpec((1, 1, H), lambda i, be_r: (be_r[i], 0, 0)),
            pl.BlockSpec((1, 1, H), lambda i, be_r: (be_r[i], 0, 0)),
            pl.BlockSpec((1, 1, H), lambda i, be_r: (be_r[i], 0, 0)),
            pl.BlockSpec((1, H, H // 2), lambda i, be_r: (be_r[i], 0, 0)),
            pl.BlockSpec((1, 1, H // 2), lambda i, be_r: (be_r[i], 0, 0)),
            pl.BlockSpec((1, H // 2, H // 4), lambda i, be_r: (be_r[i], 0, 0)),
            pl.BlockSpec((1, 1, H // 4), lambda i, be_r: (be_r[i], 0, 0)),
            pl.BlockSpec((1, 1, H // 4), lambda i, be_r: (be_r[i], 0, 0)),
            pl.BlockSpec((1, 1, H // 4), lambda i, be_r: (be_r[i], 0, 0)),
        ],
        out_specs=pl.BlockSpec((BLK, H // 4), lambda i, be_r: (i, 0)),
    )
    os_ = pl.pallas_call(
        _expert_kernel,
        grid_spec=grid_spec,
        out_shape=jax.ShapeDtypeStruct((NP, H // 4), jnp.float32),
    )(be, *ew_ops)

    sel = _gather_o(os_, pos3)

    out = pl.pallas_call(
        _out_kernel,
        grid=(T // TB,),
        in_specs=[
            pl.BlockSpec((TB, H // 4), lambda i: (i, 0)),
            pl.BlockSpec((TB, H // 4), lambda i: (i + T // TB, 0)),
            pl.BlockSpec((TB, K), lambda i: (i, 0)),
            pl.BlockSpec((TB, DM), lambda i: (i, 0)),
            const_spec((H // 4, DM)),
            const_spec((1, DM)),
            const_spec((1, 1)),
        ],
        out_specs=pl.BlockSpec((TB, DM), lambda i: (i, 0)),
        out_shape=jax.ShapeDtypeStruct((T, DM), jnp.float32),
    )(sel, sel, w, id_flat, p['Wo'].astype(bf), row(p['bo']),
      p['alpha'].reshape(1, 1))
    return out.reshape(B, L, DM)


# speedup vs baseline: 1.0134x; 1.0134x over previous
import functools

import jax
import jax.numpy as jnp
from jax import lax
from jax.experimental import pallas as pl
from jax.experimental.pallas import tpu as pltpu
from jax.experimental.pallas import tpu_sc as plsc

B, L = 2, 2048
DM, DC, DK = 1024, 768, 64
D = DM + DC + DK
E, H, K = 8, 512, 2
HG = max(D // 2, 128)
T = B * L
N = T * K
TB = 512
BLK = 512
NP = N + E * BLK
NBLK = NP // BLK
NW = 32
CH = N // NW // 64
DH = 1024
D1 = 1792


def _ln(x, g, b):
    mu = x.mean(-1, keepdims=True)
    v = ((x - mu) ** 2).mean(-1, keepdims=True)
    return (x - mu) * lax.rsqrt(v + 1e-5) * g + b


def _pack_pair(hi_f, lo_f):
    uh = lax.bitcast_convert_type(hi_f, jnp.uint32)
    ul = lax.bitcast_convert_type(lo_f, jnp.uint32)
    return lax.bitcast_convert_type(
        (uh & jnp.uint32(0xFFFF0000)) | (ul >> 16), jnp.int32)


def _gate_kernel(id_ref, ct_ref, cb_ref,
                 gW1a_ref, gW1b_ref, gb1_ref, glng_ref, glnb_ref,
                 gW2_ref, gb2_ref, gW3_ref, gb3_ref,
                 xp_ref, pos_ref, be_ref, w_ref, ti_acc):
    idv = id_ref[...]
    ctv = ct_ref[...]
    cbv = cb_ref[...]
    hi_f = jnp.concatenate(
        [ctv, cbv, jnp.zeros((TB, DH - DC - DK), jnp.float32)], axis=-1)
    xp_ref[...] = _pack_pair(hi_f, idv)
    xa = jnp.concatenate([idv, ctv], axis=-1).astype(jnp.bfloat16)
    h = jnp.dot(xa, gW1a_ref[...], preferred_element_type=jnp.float32)
    h += jnp.dot(cbv.astype(jnp.bfloat16), gW1b_ref[...],
                 preferred_element_type=jnp.float32)
    h = jax.nn.relu(_ln(h + gb1_ref[...], glng_ref[...], glnb_ref[...]))
    h2 = jnp.dot(h.astype(jnp.bfloat16), gW2_ref[...],
                 preferred_element_type=jnp.float32) + gb2_ref[...]
    h2 = jax.nn.relu(h2)
    logits = jnp.dot(h2.astype(jnp.bfloat16), gW3_ref[...],
                     preferred_element_type=jnp.float32) + gb3_ref[...]
    probs = jax.nn.softmax(logits, axis=-1)
    eidx = lax.broadcasted_iota(jnp.int32, (TB, E), 1)
    i1 = jnp.argmax(probs, axis=-1).astype(jnp.int32)
    p1 = jnp.max(probs, axis=-1)
    masked = jnp.where(eidx == i1[:, None], -jnp.inf, probs)
    i2 = jnp.argmax(masked, axis=-1).astype(jnp.int32)
    p2 = jnp.max(masked, axis=-1)
    denom = p1 + p2 + 1e-8
    w_ref[...] = jnp.stack([p1 / denom, p2 / denom], axis=-1)
    i = pl.program_id(0)
    ti_acc[pl.ds(i * (TB // 128), TB // 128)] = i1.reshape(TB // 128, 128)
    ti_acc[pl.ds(T // 128 + i * (TB // 128), TB // 128)] = (
        i2.reshape(TB // 128, 128))

    @pl.when(i == T // TB - 1)
    def _route():
        _route_body(ti_acc[...], pos_ref, be_ref)


def _route_body(ei, pos_ref, be_ref):
    cU = (lax.broadcasted_iota(jnp.int32, (128, 128), 0) <
          lax.broadcasted_iota(jnp.int32, (128, 128), 1)).astype(jnp.bfloat16)
    L64 = (lax.broadcasted_iota(jnp.int32, (64, 64), 1) <
           lax.broadcasted_iota(jnp.int32, (64, 64), 0)).astype(jnp.bfloat16)
    pos_f = jnp.zeros((64, 128), jnp.float32)
    seg = jnp.int32(0)
    seg_ends = []
    for e in range(E):
        m = ei == e
        mb = m.astype(jnp.bfloat16)
        prefix = jnp.dot(mb, cU, preferred_element_type=jnp.float32)
        rowsum = prefix[:, 127:128] + m.astype(jnp.float32)[:, 127:128]
        carry = jnp.dot(L64, rowsum.astype(jnp.bfloat16),
                        preferred_element_type=jnp.float32)
        rank = prefix + carry
        pos_f = pos_f + jnp.where(m, seg.astype(jnp.float32) + rank, 0.0)
        cnt = jnp.sum(m.astype(jnp.float32)).astype(jnp.int32)
        pe = ((cnt + BLK - 1) // BLK) * BLK
        seg = seg + pe
        seg_ends.append(seg)
    lane = lax.broadcasted_iota(jnp.int32, (1, 128), 1)
    bi = lane * BLK
    be = jnp.zeros((1, 128), jnp.int32)
    for e in range(E):
        be = be + (bi >= seg_ends[e]).astype(jnp.int32)
    be_ref[...] = jnp.where(lane == NBLK, seg_ends[-1],
                            jnp.minimum(be, E - 1))
    pos_ref[...] = pos_f.astype(jnp.int32)


_vmesh = plsc.VectorSubcoreMesh(core_axis_name="c", subcore_axis_name="s")


SCH = 8


@functools.partial(
    pl.kernel, mesh=_vmesh,
    out_type=jax.ShapeDtypeStruct((NP, DH), jnp.int32),
    scratch_types=[pltpu.VMEM((SCH, 32), jnp.int32),
                   pltpu.VMEM((32, DH), jnp.int32),
                   pltpu.VMEM((32, DH), jnp.int32),
                   pltpu.SemaphoreType.DMA,
                   pltpu.SemaphoreType.DMA])
def _scatter_x(x_hbm, pos_hbm, xs_hbm, pos_v, xbuf0, xbuf1, sem0, sem1):
    wid = lax.axis_index("s") * 2 + lax.axis_index("c")
    t0 = (wid * 256) % T
    pltpu.sync_copy(pos_hbm.at[wid], pos_v)
    bufs = (xbuf0, xbuf1)
    sems = (sem0, sem1)
    pending = [None, None]
    for q in range(SCH):
        b = bufs[q % 2]
        if pending[q % 2] is not None:
            pending[q % 2].wait()
        pltpu.sync_copy(x_hbm.at[pl.ds(t0 + q * 32, 32)], b)
        pending[q % 2] = pltpu.async_copy(b, xs_hbm.at[pos_v.at[q]],
                                          sems[q % 2])
    pending[0].wait()
    pending[1].wait()


@functools.partial(
    pl.kernel, mesh=_vmesh,
    out_type=jax.ShapeDtypeStruct((N, H // 4), jnp.float32),
    scratch_types=[pltpu.VMEM((CH, 64), jnp.int32),
                   pltpu.VMEM((64, H // 4), jnp.float32),
                   pltpu.SemaphoreType.DMA])
def _gather_o(os_hbm, pos_hbm, sel_hbm, pos_v, obuf, sem):
    wid = lax.axis_index("s") * 2 + lax.axis_index("c")
    base = wid * 256
    pltpu.sync_copy(pos_hbm.at[wid], pos_v)
    for c in range(CH):
        pltpu.async_copy(os_hbm.at[pos_v.at[c]], obuf, sem).wait()
        pltpu.sync_copy(obuf, sel_hbm.at[pl.ds(base + c * 64, 64)])


def _expert_kernel(be_ref, xs_ref, eW1a_ref, eW1b_ref, eb1_ref,
                   g1_ref, b1_ref, eW2_ref, eb2_ref, eW3_ref, eb3_ref,
                   g2_ref, b2_ref, os_ref):
    @pl.when(pl.program_id(0) * BLK < be_ref[NBLK])
    def _body():
        xi = lax.bitcast_convert_type(xs_ref[...], jnp.uint32)
        lo = lax.bitcast_convert_type(xi << 16, jnp.float32)
        hi = lax.bitcast_convert_type(xi & jnp.uint32(0xFFFF0000), jnp.float32)
        xa = lo.astype(jnp.bfloat16)
        xb = hi.astype(jnp.bfloat16)
        a = jnp.dot(xa, eW1a_ref[0], preferred_element_type=jnp.float32)
        a += jnp.dot(xb, eW1b_ref[0], preferred_element_type=jnp.float32)
        a = jax.nn.relu(_ln(a + eb1_ref[0], g1_ref[0], b1_ref[0]))
        b2v = jnp.dot(a.astype(jnp.bfloat16), eW2_ref[0],
                      preferred_element_type=jnp.float32) + eb2_ref[0]
        b2v = jax.nn.relu(b2v)
        o = jnp.dot(b2v.astype(jnp.bfloat16), eW3_ref[0],
                    preferred_element_type=jnp.float32) + eb3_ref[0]
        os_ref[...] = _ln(o, g2_ref[0], b2_ref[0])


def _out_kernel(sel0_ref, sel1_ref, w_ref, id_ref, Wo_ref, bo_ref,
                alpha_ref, out_ref):
    w = w_ref[...]
    fused = sel0_ref[...] * w[:, 0:1] + sel1_ref[...] * w[:, 1:2]
    proj = jnp.dot(fused.astype(jnp.bfloat16), Wo_ref[...],
                   preferred_element_type=jnp.float32) + bo_ref[...]
    out_ref[...] = id_ref[...] + alpha_ref[0, 0] * proj


def kernel(id_emb, content_emb, collab_emb, params):
    p = params
    bf = jnp.bfloat16
    row = lambda a: a.reshape(1, -1)
    id_flat = id_emb.reshape(T, DM)
    ct_flat = content_emb.reshape(T, DC)
    cb_flat = collab_emb.reshape(T, DK)

    def const_spec(shape):
        return pl.BlockSpec(shape, lambda i: (0,) * len(shape))

    gW1a = p['gW1'][:D1].astype(bf)
    gW1b = p['gW1'][D1:].astype(bf)
    gate_ops = [id_flat, ct_flat, cb_flat, gW1a, gW1b, row(p['gb1']),
                row(p['gln_g']), row(p['gln_b']), p['gW2'].astype(bf),
                row(p['gb2']), p['gW3'].astype(bf), row(p['gb3'])]
    xp, pos64, be128, w = pl.pallas_call(
        _gate_kernel,
        grid=(T // TB,),
        in_specs=[pl.BlockSpec((TB, DM), lambda i: (i, 0)),
                  pl.BlockSpec((TB, DC), lambda i: (i, 0)),
                  pl.BlockSpec((TB, DK), lambda i: (i, 0))] +
                 [const_spec(op.shape) for op in gate_ops[3:]],
        out_specs=[pl.BlockSpec((TB, DH), lambda i: (i, 0)),
                   const_spec((64, 128)),
                   const_spec((1, 128)),
                   pl.BlockSpec((TB, K), lambda i: (i, 0))],
        out_shape=[jax.ShapeDtypeStruct((T, DH), jnp.int32),
                   jax.ShapeDtypeStruct((64, 128), jnp.int32),
                   jax.ShapeDtypeStruct((1, 128), jnp.int32),
                   jax.ShapeDtypeStruct((T, K), jnp.float32)],
        scratch_shapes=[pltpu.VMEM((64, 128), jnp.int32)],
    )(*gate_ops)
    pos3 = pos64.reshape(NW, CH, 64)
    be = be128[0, :NBLK + 1]

    xs = _scatter_x(xp, pos64.reshape(NW, SCH, 32))

    e3 = lambda a: a.reshape(E, 1, -1)
    eW1a = p['eW1'][:, :DM].astype(bf)
    eW1b = jnp.concatenate(
        [p['eW1'][:, DM:].astype(bf), jnp.zeros((E, DH - DC - DK, H), bf)],
        axis=1)
    ew_ops = [xs, eW1a, eW1b, e3(p['eb1']), e3(p['eln1_g']), e3(p['eln1_b']),
              p['eW2'].astype(bf), e3(p['eb2']), p['eW3'].astype(bf),
              e3(p['eb3']), e3(p['eln2_g']), e3(p['eln2_b'])]
    grid_spec = pltpu.PrefetchScalarGridSpec(
        num_scalar_prefetch=1,
        grid=(NBLK,),
        in_specs=[
            pl.BlockSpec((BLK, DH), lambda i, be_r: (i, 0)),
            pl.BlockSpec((1, DH, H), lambda i, be_r: (be_r[i], 0, 0)),
            pl.BlockSpec((1, DH, H), lambda i, be_r: (be_r[i], 0, 0)),
            pl.BlockSpec((1, 1, H), lambda i, be_r: (be_r[i], 0, 0)),
            pl.BlockSpec((1, 1, H), lambda i, be_r: (be_r[i], 0, 0)),
            pl.BlockSpec((1, 1, H), lambda i, be_r: (be_r[i], 0, 0)),
            pl.BlockSpec((1, H, H // 2), lambda i, be_r: (be_r[i], 0, 0)),
            pl.BlockSpec((1, 1, H // 2), lambda i, be_r: (be_r[i], 0, 0)),
            pl.BlockSpec((1, H // 2, H // 4), lambda i, be_r: (be_r[i], 0, 0)),
            pl.BlockSpec((1, 1, H // 4), lambda i, be_r: (be_r[i], 0, 0)),
            pl.BlockSpec((1, 1, H // 4), lambda i, be_r: (be_r[i], 0, 0)),
            pl.BlockSpec((1, 1, H // 4), lambda i, be_r: (be_r[i], 0, 0)),
        ],
        out_specs=pl.BlockSpec((BLK, H // 4), lambda i, be_r: (i, 0)),
    )
    os_ = pl.pallas_call(
        _expert_kernel,
        grid_spec=grid_spec,
        out_shape=jax.ShapeDtypeStruct((NP, H // 4), jnp.float32),
    )(be, *ew_ops)

    sel = _gather_o(os_, pos3)

    out = pl.pallas_call(
        _out_kernel,
        grid=(T // TB,),
        in_specs=[
            pl.BlockSpec((TB, H // 4), lambda i: (i, 0)),
            pl.BlockSpec((TB, H // 4), lambda i: (i + T // TB, 0)),
            pl.BlockSpec((TB, K), lambda i: (i, 0)),
            pl.BlockSpec((TB, DM), lambda i: (i, 0)),
            const_spec((H // 4, DM)),
            const_spec((1, DM)),
            const_spec((1, 1)),
        ],
        out_specs=pl.BlockSpec((TB, DM), lambda i: (i, 0)),
        out_shape=jax.ShapeDtypeStruct((T, DM), jnp.float32),
    )(sel, sel, w, id_flat, p['Wo'].astype(bf), row(p['bo']),
      p['alpha'].reshape(1, 1))
    return out.reshape(B, L, DM)
